# baseline (speedup 1.0000x reference)
import functools

import jax
import jax.numpy as jnp
from jax import lax
from jax.experimental import pallas as pl
from jax.experimental.pallas import tpu as pltpu
from jax.experimental.pallas import tpu_sc as plsc

N = 10000
E = 320000
D = 128
R = 4
B = 512
NC = 2
NS = 16
L = 16
NW = NC * NS
EPT = E // NW
C = 80
NCHUNK = EPT // C
ROW_STRIDE = 624
ROW_SPAN = 640
CNT_SLICE = (4 * N) // 10
EPC = E // NS
NCHUNK_CNT = EPC // C

_mesh = plsc.VectorSubcoreMesh(core_axis_name="c", subcore_axis_name="s")


@functools.partial(
    pl.kernel,
    mesh=_mesh,
    out_type=jax.ShapeDtypeStruct((4 * N,), jnp.float32),
    scratch_types=[
        pltpu.VMEM((NCHUNK_CNT, C), jnp.int32),
        pltpu.VMEM((C,), jnp.float32),
        pltpu.VMEM((CNT_SLICE,), jnp.float32),
        pltpu.VMEM_SHARED((4 * N,), jnp.float32),
    ],
)
def _count_kernel(key_hbm, inv_hbm, keys_v, ones_v, buf_v, cnt_sh):
    s = lax.axis_index("s")

    def zero_body(i, _):
        buf_v[pl.ds(i * L, L)] = jnp.zeros((L,), jnp.float32)
        return 0

    lax.fori_loop(0, CNT_SLICE // L, zero_body, 0)

    @pl.when(s < 10)
    def _():
        pltpu.sync_copy(buf_v, cnt_sh.at[pl.ds(s * CNT_SLICE, CNT_SLICE)])

    for i in range(C // L):
        ones_v[pl.ds(i * L, L)] = jnp.ones((L,), jnp.float32)
    pltpu.sync_copy(key_hbm.at[s], keys_v)
    plsc.subcore_barrier()

    def chunk(k, _):
        pltpu.sync_copy(ones_v, cnt_sh.at[keys_v.at[k]], add=True)
        return 0

    lax.fori_loop(0, NCHUNK_CNT, chunk, 0)
    plsc.subcore_barrier()

    @pl.when(s < 10)
    def _():
        pltpu.sync_copy(cnt_sh.at[pl.ds(s * CNT_SLICE, CNT_SLICE)], buf_v)

        def inv_body(i, _):
            sl = pl.ds(i * L, L)
            buf_v[sl] = jnp.maximum(buf_v[sl], 1.0)
            return 0

        lax.fori_loop(0, CNT_SLICE // L, inv_body, 0)
        pltpu.sync_copy(buf_v, inv_hbm.at[pl.ds(s * CNT_SLICE, CNT_SLICE)])


MAXE = 6400
TRASH = N


@functools.partial(
    pl.kernel,
    mesh=_mesh,
    out_type=jax.ShapeDtypeStruct((NC, 2, N, D), jnp.float32),
    scratch_types=[
        pltpu.VMEM((MAXE + 8,), jnp.int32),
        pltpu.VMEM((L,), jnp.int32),
        pltpu.VMEM((C,), jnp.int32), pltpu.VMEM((C,), jnp.int32),
        pltpu.VMEM((C,), jnp.int32), pltpu.VMEM((C,), jnp.int32),
        pltpu.VMEM((C, D), jnp.float32), pltpu.VMEM((C, D), jnp.float32),
        pltpu.VMEM_SHARED((N + L, D), jnp.float32),
        pltpu.SemaphoreType.DMA, pltpu.SemaphoreType.DMA,
        pltpu.SemaphoreType.DMA, pltpu.SemaphoreType.DMA,
    ],
)
def _edge_kernel2(h_hbm, pidx_hbm, work_hbm, out_hbm,
                  pidx_v, work_v, src0, src1, dst0, dst1,
                  rows0, rows1, acc_sh, g0, g1, s0, s1):
    c = lax.axis_index("c")
    s = lax.axis_index("s")
    SRC = (src0, src1)
    DST = (dst0, dst1)
    ROWS = (rows0, rows1)
    G = (g0, g1)
    SS = (s0, s1)

    def zero_rows(rows):
        def zb(i, _):
            rows[i >> 3, pl.ds((i & 7) * L, L)] = jnp.zeros((L,), jnp.float32)
            return 0

        lax.fori_loop(0, C * (D // L), zb, 0)

    def run_phase(p):
        pltpu.sync_copy(work_hbm.at[c, p, s], work_v)
        w16 = work_v[...]
        astart = pl.multiple_of(w16[0], 8)
        lo = w16[1]
        ec = w16[2]
        nch = (ec + C - 1) // C

        zero_rows(rows0)
        for t in range(ROW_SPAN // C):
            pltpu.sync_copy(rows0,
                            acc_sh.at[pl.ds(s * ROW_STRIDE + t * C, C)])

        @pl.when(s == 0)
        def _():
            pltpu.sync_copy(rows0.at[pl.ds(0, L)], acc_sh.at[pl.ds(N, L)])

        pltpu.sync_copy(pidx_hbm.at[pl.ds(astart, MAXE + 8)], pidx_v)
        plsc.subcore_barrier()

        def unpack(k, b):
            def group(g, _):
                p16 = pidx_v[pl.ds(lo + k * C + g * L, L)]
                lane = g * L + lax.iota(jnp.int32, L)
                valid = lane < (ec - k * C)
                d16 = lax.bitwise_and(p16, (1 << 14) - 1)
                r16 = lax.shift_right_logical(p16, 14)
                SRC[b][pl.ds(g * L, L)] = jnp.where(valid, r16, 0)
                DST[b][pl.ds(g * L, L)] = jnp.where(
                    valid, d16, TRASH + lax.iota(jnp.int32, L))
                return 0

            lax.fori_loop(0, C // L, group, 0)

        def launch(k, b):
            unpack(k, b)
            pltpu.async_copy(h_hbm.at[SRC[b]], ROWS[b], G[b])

        def wait_g(b):
            pltpu.make_async_copy(h_hbm.at[SRC[b]], ROWS[b], G[b]).wait()

        def wait_s(b):
            pltpu.make_async_copy(ROWS[b], acc_sh.at[DST[b]], SS[b]).wait()

        def scat(b):
            pltpu.async_copy(ROWS[b], acc_sh.at[DST[b]], SS[b], add=True)

        @pl.when(nch > 0)
        def _():
            launch(0, 0)

        def pair(m, _):
            k0 = 2 * m
            k1 = k0 + 1

            @pl.when(jnp.logical_and(m > 0, k1 < nch + 2))
            def _():
                wait_s(1)

            @pl.when(k1 < nch)
            def _():
                launch(k1, 1)

            wait_g(0)
            scat(0)
            wait_s(0)

            @pl.when(k0 + 2 < nch)
            def _():
                launch(k0 + 2, 0)

            @pl.when(k1 < nch)
            def _():
                wait_g(1)
                scat(1)

            return 0

        lax.fori_loop(0, (nch + 1) // 2, pair, 0)

        @pl.when(jnp.logical_and(nch > 1, (nch % 2) == 0))
        def _():
            wait_s(1)

        plsc.subcore_barrier()
        pltpu.sync_copy(acc_sh.at[pl.ds(s * ROW_STRIDE, ROW_SPAN)],
                        out_hbm.at[c, p, pl.ds(s * ROW_STRIDE, ROW_SPAN)])
        plsc.subcore_barrier()

    run_phase(0)
    run_phase(1)


@functools.partial(
    pl.kernel,
    mesh=_mesh,
    out_type=jax.ShapeDtypeStruct((NC, N, D), jnp.float32),
    scratch_types=[
        pltpu.VMEM((NCHUNK, C), jnp.int32),
        pltpu.VMEM((C,), jnp.int32), pltpu.VMEM((C,), jnp.int32),
        pltpu.VMEM((C,), jnp.int32),
        pltpu.VMEM((C,), jnp.int32), pltpu.VMEM((C,), jnp.int32),
        pltpu.VMEM((C,), jnp.int32),
        pltpu.VMEM((C,), jnp.int32), pltpu.VMEM((C,), jnp.int32),
        pltpu.VMEM((C,), jnp.int32),
        pltpu.VMEM((C,), jnp.float32), pltpu.VMEM((C,), jnp.float32),
        pltpu.VMEM((C,), jnp.float32),
        pltpu.VMEM((C, D), jnp.float32), pltpu.VMEM((C, D), jnp.float32),
        pltpu.VMEM((C, D), jnp.float32),
        pltpu.VMEM_SHARED((N, D), jnp.float32),
        pltpu.SemaphoreType.DMA, pltpu.SemaphoreType.DMA,
        pltpu.SemaphoreType.DMA,
        pltpu.SemaphoreType.DMA, pltpu.SemaphoreType.DMA,
        pltpu.SemaphoreType.DMA,
        pltpu.SemaphoreType.DMA, pltpu.SemaphoreType.DMA,
        pltpu.SemaphoreType.DMA,
    ],
)
def _edge_kernel(y_hbm, pidx_hbm, inv_hbm, out_hbm,
                 pidx_v, ridx0, ridx1, ridx2, dst0, dst1, dst2,
                 key0, key1, key2, svl0, svl1, svl2,
                 rows0, rows1, rows2, acc_sh,
                 g0, g1, g2, v0, v1, v2, s0, s1, s2):
    c = lax.axis_index("c")
    s = lax.axis_index("s")
    wid = c * NS + s
    RIDX = (ridx0, ridx1, ridx2)
    DST = (dst0, dst1, dst2)
    KEY = (key0, key1, key2)
    SVL = (svl0, svl1, svl2)
    ROWS = (rows0, rows1, rows2)
    G = (g0, g1, g2)
    V = (v0, v1, v2)
    SS = (s0, s1, s2)

    def zero_body(i, _):
        rows0[i >> 3, pl.ds((i & 7) * L, L)] = jnp.zeros((L,), jnp.float32)
        return 0

    lax.fori_loop(0, C * (D // L), zero_body, 0)
    for t in range(ROW_SPAN // C):
        pltpu.sync_copy(rows0,
                        acc_sh.at[pl.ds(s * ROW_STRIDE + t * C, C)])

    pltpu.sync_copy(pidx_hbm.at[wid], pidx_v)
    plsc.subcore_barrier()

    def launch(k, b):
        def group(g, _):
            sl = pl.ds(g * L, L)
            p16 = pidx_v[k, sl]
            d16 = lax.bitwise_and(p16, (1 << 14) - 1)
            r16 = lax.shift_right_logical(p16, 14)
            RIDX[b][sl] = r16
            DST[b][sl] = d16
            KEY[b][sl] = 4 * d16 + lax.bitwise_and(r16, 3)
            return 0

        lax.fori_loop(0, C // L, group, 0)
        pltpu.async_copy(y_hbm.at[RIDX[b]], ROWS[b], G[b])
        pltpu.async_copy(inv_hbm.at[KEY[b]], SVL[b], V[b])

    def wait_g(b):
        pltpu.make_async_copy(y_hbm.at[RIDX[b]], ROWS[b], G[b]).wait()

    def wait_v(b):
        pltpu.make_async_copy(inv_hbm.at[KEY[b]], SVL[b], V[b]).wait()

    def wait_s(b):
        pltpu.make_async_copy(ROWS[b], acc_sh.at[DST[b]], SS[b]).wait()

    def scale(b):
        def group(g, _):
            sv16 = SVL[b][pl.ds(g * L, L)]
            for e in range(L):
                sb = jnp.broadcast_to(sv16[e], (L,))
                row = g * L + e
                for j in range(D // L):
                    sl = pl.ds(j * L, L)
                    ROWS[b][row, sl] = ROWS[b][row, sl] * sb
            return 0

        lax.fori_loop(0, C // L, group, 0)

    def step(j, b, bn, first_steps):
        if not first_steps:
            wait_s(bn)

        @pl.when(j + 1 < NCHUNK)
        def _():
            launch(j + 1, bn)

        wait_g(b)
        wait_v(b)
        scale(b)
        pltpu.async_copy(ROWS[b], acc_sh.at[DST[b]], SS[b], add=True)

    launch(0, 0)

    def triple(m, _):
        j = 3 * m
        for jj in range(3):
            step(j + jj, jj, (jj + 1) % 3, False)
        return 0

    for jj in range(3):
        step(jj, jj, (jj + 1) % 3, jj < 2)
    lax.fori_loop(1, (NCHUNK - 2) // 3, triple, 0)

    wait_s(1)
    launch(NCHUNK - 1, 1)
    wait_g(0)
    wait_v(0)
    scale(0)
    pltpu.async_copy(ROWS[0], acc_sh.at[DST[0]], SS[0], add=True)

    wait_s(2)
    wait_g(1)
    wait_v(1)
    scale(1)
    pltpu.sync_copy(ROWS[1], acc_sh.at[DST[1]], add=True)
    wait_s(0)

    plsc.subcore_barrier()
    pltpu.sync_copy(acc_sh.at[pl.ds(s * ROW_STRIDE, ROW_SPAN)],
                    out_hbm.at[c, pl.ds(s * ROW_STRIDE, ROW_SPAN)])


POOL_CHUNKS = N // C
POOL_ITERS = -(-POOL_CHUNKS // NW)
B_PER_TILE = B // NS


@functools.partial(
    pl.kernel,
    mesh=_mesh,
    out_type=jax.ShapeDtypeStruct((NC, B, D), jnp.float32),
    scratch_types=[
        pltpu.VMEM((C,), jnp.int32),
        pltpu.VMEM((C, D), jnp.float32),
        pltpu.VMEM_SHARED((B, D), jnp.float32),
    ],
)
def _pool_kernel(h_hbm, batch_hbm, out_hbm, bat_v, rows_v, acc_sh):
    c = lax.axis_index("c")
    s = lax.axis_index("s")
    wid = c * NS + s

    def zero_body(i, _):
        rows_v[i >> 3, pl.ds((i & 7) * L, L)] = jnp.zeros((L,), jnp.float32)
        return 0

    lax.fori_loop(0, B_PER_TILE * (D // L), zero_body, 0)
    pltpu.sync_copy(rows_v.at[pl.ds(0, B_PER_TILE)],
                    acc_sh.at[pl.ds(s * B_PER_TILE, B_PER_TILE)])
    plsc.subcore_barrier()

    def body(k, _):
        cid = k * NW + wid

        @pl.when(cid < POOL_CHUNKS)
        def _():
            base = cid * C
            pltpu.sync_copy(batch_hbm.at[pl.ds(base, C)], bat_v)
            pltpu.sync_copy(h_hbm.at[pl.ds(base, C)], rows_v)
            pltpu.sync_copy(rows_v, acc_sh.at[bat_v], add=True)

        return 0

    lax.fori_loop(0, POOL_ITERS, body, 0)
    plsc.subcore_barrier()
    pltpu.sync_copy(acc_sh.at[pl.ds(s * B_PER_TILE, B_PER_TILE)],
                    out_hbm.at[c, pl.ds(s * B_PER_TILE, B_PER_TILE)])


BLK = 1000
GRID = N // BLK


def _silu(v):
    return v / (1.0 + jnp.exp(-v))


def _mm(a, w):
    return jnp.dot(a, w, preferred_element_type=jnp.float32)


def _k0_body(x_ref, wr_ref, b_ref, pre_ref):
    pre_ref[...] = _mm(x_ref[...], wr_ref[...]) + b_ref[...]


def _agg_sum(pre, a_refs, inv_ref, w_refs):
    acc = pre
    for r in range(R):
        scaled = a_refs[r][...] / inv_ref[...][:, r:r + 1]
        acc = acc + _mm(scaled, w_refs[r][...])
    return acc


def _k1_body(pr_ref, a0, a1, a2, a3, inv_ref, w0, w1, w2, w3,
             g_ref, bb_ref, wr_ref, b1_ref,
             h_ref, pr2_ref, pre_buf, acc_ref):
    p = pl.program_id(0)
    i = pl.program_id(1)

    @pl.when(jnp.logical_and(p == 0, i == 0))
    def _():
        acc_ref[...] = jnp.zeros_like(acc_ref)

    @pl.when(p == 0)
    def _():
        v = _agg_sum(pr_ref[...], (a0, a1, a2, a3), inv_ref,
                     (w0, w1, w2, w3))
        pre_buf[pl.ds(i * BLK, BLK), :] = v
        acc_ref[0:1] = acc_ref[0:1] + jnp.sum(v, axis=0, keepdims=True)

    @pl.when(p == 1)
    def _():
        m = acc_ref[0:1] / N
        d = pre_buf[pl.ds(i * BLK, BLK), :] - m

        @pl.when(i == 0)
        def _():
            acc_ref[1:2] = jnp.zeros((1, D), jnp.float32)

        acc_ref[1:2] = acc_ref[1:2] + jnp.sum(d * d, axis=0, keepdims=True)

    @pl.when(p == 2)
    def _():
        m = acc_ref[0:1] / N
        var = acc_ref[1:2] / N
        v = ((pre_buf[pl.ds(i * BLK, BLK), :] - m) / jnp.sqrt(var + 1e-5)
             * g_ref[...] + bb_ref[...])
        h = _silu(v)
        h_ref[...] = h
        pr2_ref[...] = _mm(h, wr_ref[...]) + b1_ref[...]


def _k2_body(pr_ref, a0, a1, a2, a3, inv_ref, w0, w1, w2, w3,
             wr_ref, b_ref, h_ref, pr2_ref):
    h = _silu(_agg_sum(pr_ref[...], (a0, a1, a2, a3), inv_ref,
                       (w0, w1, w2, w3)))
    h_ref[...] = h
    pr2_ref[...] = _mm(h, wr_ref[...]) + b_ref[...]


def _k3_body(pr_ref, a0, a1, a2, a3, inv_ref, w0, w1, w2, w3, h_ref):
    h_ref[...] = _silu(_agg_sum(pr_ref[...], (a0, a1, a2, a3), inv_ref,
                                (w0, w1, w2, w3)))


def _head_body(hg0_ref, hg1_ref, mf_ref, wm0_ref, bm0_ref, gm_ref, bbm_ref,
               wm1_ref, bm1_ref, wf0_ref, bf0_ref, wf1_ref, bf1_ref,
               wf2_ref, bf2_ref, out_ref):
    hg = hg0_ref[...] + hg1_ref[...]
    hm = _mm(mf_ref[...], wm0_ref[...]) + bm0_ref[...]
    m = jnp.mean(hm, axis=0, keepdims=True)
    var = jnp.mean((hm - m) ** 2, axis=0, keepdims=True)
    hm = (hm - m) / jnp.sqrt(var + 1e-5) * gm_ref[...] + bbm_ref[...]
    hm = _silu(hm)
    hm = _silu(_mm(hm, wm1_ref[...]) + bm1_ref[...])
    z = _silu(_mm(hg, wf0_ref[0:D, :]) + _mm(hm, wf0_ref[D:2 * D, :]) + bf0_ref[...])
    z = _silu(_mm(z, wf1_ref[...]) + bf1_ref[...])
    out_ref[...] = _mm(z, wf2_ref[...]) + bf2_ref[...]


def _row_spec(w):
    return pl.BlockSpec((BLK, w), lambda i: (i, 0))


def _full_spec(shape):
    nd = len(shape)
    return pl.BlockSpec(shape, lambda i: (0,) * nd)


def _k0(x, wr, b):
    return pl.pallas_call(
        _k0_body,
        grid=(GRID,),
        in_specs=[_row_spec(D), _full_spec((D, D)), _full_spec((1, D))],
        out_specs=_row_spec(D),
        out_shape=jax.ShapeDtypeStruct((N, D), jnp.float32),
    )(x, wr, b)


def _row_spec2(w):
    return pl.BlockSpec((BLK, w), lambda p, i: (jnp.where(p == 0, i, 0), 0))


def _out_spec2(w):
    return pl.BlockSpec((BLK, w), lambda p, i: (jnp.where(p == 2, i, 0), 0))


def _full_spec2(shape):
    nd = len(shape)
    return pl.BlockSpec(shape, lambda p, i: (0,) * nd)


def _k1(pr, aggs, invn, ws, g, bb, wr, b1):
    return pl.pallas_call(
        _k1_body,
        grid=(3, GRID),
        in_specs=[_row_spec2(D), _row_spec2(D), _row_spec2(D), _row_spec2(D),
                  _row_spec2(D), _row_spec2(R),
                  _full_spec2((D, D)), _full_spec2((D, D)),
                  _full_spec2((D, D)), _full_spec2((D, D)),
                  _full_spec2((1, D)), _full_spec2((1, D)),
                  _full_spec2((D, D)), _full_spec2((1, D))],
        out_specs=[_out_spec2(D), _out_spec2(D)],
        out_shape=[jax.ShapeDtypeStruct((N, D), jnp.float32),
                   jax.ShapeDtypeStruct((N, D), jnp.float32)],
        scratch_shapes=[pltpu.VMEM((N, D), jnp.float32),
                        pltpu.VMEM((2, D), jnp.float32)],
    )(pr, *aggs, invn, *ws, g, bb, wr, b1)


def _k2(pr, aggs, invn, ws, wr, b):
    return pl.pallas_call(
        _k2_body,
        grid=(GRID,),
        in_specs=[_row_spec(D), _row_spec(D), _row_spec(D), _row_spec(D),
                  _row_spec(D), _row_spec(R),
                  _full_spec((D, D)), _full_spec((D, D)),
                  _full_spec((D, D)), _full_spec((D, D)),
                  _full_spec((D, D)), _full_spec((1, D))],
        out_specs=[_row_spec(D), _row_spec(D)],
        out_shape=[jax.ShapeDtypeStruct((N, D), jnp.float32),
                   jax.ShapeDtypeStruct((N, D), jnp.float32)],
    )(pr, *aggs, invn, *ws, wr, b)


def _k3(pr, aggs, invn, ws):
    return pl.pallas_call(
        _k3_body,
        grid=(GRID,),
        in_specs=[_row_spec(D), _row_spec(D), _row_spec(D), _row_spec(D),
                  _row_spec(D), _row_spec(R),
                  _full_spec((D, D)), _full_spec((D, D)),
                  _full_spec((D, D)), _full_spec((D, D))],
        out_specs=_row_spec(D),
        out_shape=jax.ShapeDtypeStruct((N, D), jnp.float32),
    )(pr, *aggs, invn, *ws)


def _head(hg0, hg1, mf, pm, bn_m, pf):
    MF = mf.shape[1]
    args = (hg0, hg1, mf,
            pm[0]["w"], pm[0]["b"].reshape(1, -1),
            bn_m["g"].reshape(1, -1), bn_m["b"].reshape(1, -1),
            pm[1]["w"], pm[1]["b"].reshape(1, -1),
            pf[0]["w"], pf[0]["b"].reshape(1, -1),
            pf[1]["w"], pf[1]["b"].reshape(1, -1),
            pf[2]["w"], pf[2]["b"].reshape(1, -1))
    return pl.pallas_call(
        _head_body,
        out_shape=jax.ShapeDtypeStruct((B, 1), jnp.float32),
    )(*args)


def kernel(x, edge_index, edge_type, batch, mol_feats, params):
    x = x.astype(jnp.float32)
    src = edge_index[0].astype(jnp.int32)
    dst = edge_index[1].astype(jnp.int32)
    et = edge_type.astype(jnp.int32)
    batch = batch.astype(jnp.int32)

    key = R * dst + et
    pidx = lax.shift_left(src, 14) | dst
    pidx_pad = jnp.concatenate(
        [pidx, jnp.zeros((MAXE + L,), jnp.int32)])

    off = jnp.searchsorted(et, jnp.arange(R + 1, dtype=jnp.int32))
    off = off.astype(jnp.int32)
    rlen = off[1:] - off[:-1]
    stride = ((rlen + NS - 1) // NS + 7) // 8 * 8
    t_ar = jnp.arange(NS, dtype=jnp.int32)
    start = off[:-1][:, None] + t_ar[None, :] * stride[:, None]
    ec = jnp.clip(rlen[:, None] - t_ar[None, :] * stride[:, None],
                  0, stride[:, None])
    astart = start & ~7
    lo = start - astart
    table = jnp.stack(
        [astart, lo, ec] + [jnp.zeros_like(ec)] * (L - 3),
        axis=-1)
    work = table.reshape(2, 2, NS, L).transpose(1, 0, 2, 3)

    gc = params["gc"]
    b_ = [p["b"].reshape(1, D) for p in gc]
    wrel = [[p["w_rel"][r] for r in range(R)] for p in gc]

    inv = _count_kernel(key.reshape(NS, NCHUNK_CNT, C))
    invn = inv.reshape(N, R)

    def quads(a):
        return tuple(a[r % 2, r // 2] for r in range(R))

    pre_root1 = _k0(x, gc[0]["w_root"], b_[0])
    agg1 = _edge_kernel2(x, pidx_pad, work)
    h1, pre_root2 = _k1(pre_root1, quads(agg1), invn, wrel[0],
                        params["bn_gc"]["g"].reshape(1, D),
                        params["bn_gc"]["b"].reshape(1, D),
                        gc[1]["w_root"], b_[1])

    agg2 = _edge_kernel2(h1, pidx_pad, work)
    h2, pre_root3 = _k2(pre_root2, quads(agg2), invn, wrel[1],
                        gc[2]["w_root"], b_[2])

    agg3 = _edge_kernel2(h2, pidx_pad, work)
    h3 = _k3(pre_root3, quads(agg3), invn, wrel[2])

    hg = _pool_kernel(h3, batch)
    return _head(hg[0], hg[1], mol_feats.astype(jnp.float32),
                 params["fc_m"], params["bn_m"], params["fc"])
